# baseline (speedup 1.0000x reference)
import functools

import jax
import jax.numpy as jnp
from jax import lax
from jax.experimental import pallas as pl
from jax.experimental.pallas import tpu as pltpu
from jax.experimental.pallas import tpu_sc as plsc

_NCORE = 2
_NSUB = 16
_NW = _NCORE * _NSUB
_LANE = 16

_CH = 128
_NPAD = 10240
_EPAD = 327680

_MM = dict(preferred_element_type=jnp.float32, precision=lax.Precision.HIGHEST)


def _sc_degree(dst, zeros, ones):
    (e,) = dst.shape
    npad, h = zeros.shape
    epw = e // _NW
    nchunk = epw // _CH
    rps = npad // _NSUB

    mesh = plsc.VectorSubcoreMesh(
        core_axis_name="c", subcore_axis_name="s",
        num_cores=_NCORE, num_subcores=_NSUB)

    @functools.partial(
        pl.kernel,
        out_type=jax.ShapeDtypeStruct((_NCORE, npad, h), jnp.float32),
        mesh=mesh,
        scratch_types=[
            pltpu.VMEM_SHARED((npad, h), jnp.float32),
            pltpu.VMEM((_CH,), jnp.int32),
            pltpu.VMEM((_CH,), jnp.int32),
            pltpu.VMEM((_CH, h), jnp.float32),
            pltpu.SemaphoreType.DMA,
            pltpu.SemaphoreType.DMA,
        ],
    )
    def kern(dst_h, zeros_h, ones_h, out_h, acc, dstb0, dstb1, onesv,
             sd0, sd1):
        c = lax.axis_index("c")
        s = lax.axis_index("s")
        wid = c * _NSUB + s
        pltpu.sync_copy(zeros_h.at[pl.ds(s * rps, rps)],
                        acc.at[pl.ds(s * rps, rps)])
        pltpu.sync_copy(ones_h, onesv)
        plsc.subcore_barrier()

        dstb = (dstb0, dstb1)
        sd = (sd0, sd1)

        def dst_load(i, p, wait):
            cp = pltpu.make_async_copy(dst_h.at[wid, i], dstb[p], sd[p])
            cp.wait() if wait else cp.start()

        dst_load(0, 0, False)
        dst_load(1, 1, False)

        def step(i, p, loads):
            dst_load(i, p, True)
            pltpu.sync_copy(onesv, acc.at[dstb[p]], add=True)
            if loads:
                dst_load(i + 2, p, False)

        def pair(g, carry):
            step(2 * g, 0, True)
            step(2 * g + 1, 1, True)
            return carry

        lax.fori_loop(0, (nchunk - 2) // 2, pair, 0)
        step(nchunk - 2, 0, False)
        step(nchunk - 1, 1, False)

        plsc.subcore_barrier()
        pltpu.sync_copy(acc.at[pl.ds(s * rps, rps)],
                        out_h.at[c, pl.ds(s * rps, rps)])

    dst3 = dst.reshape(_NW, nchunk, _CH)
    return kern(dst3, zeros, ones)


def _sc_propagate(table, src, dst, zeros):
    _, h = table.shape
    npad = zeros.shape[0]
    (e,) = src.shape
    epw = e // _NW
    nchunk = epw // _CH
    rps = npad // _NSUB

    mesh = plsc.VectorSubcoreMesh(
        core_axis_name="c", subcore_axis_name="s",
        num_cores=_NCORE, num_subcores=_NSUB)

    @functools.partial(
        pl.kernel,
        out_type=jax.ShapeDtypeStruct((_NCORE, npad, h), jnp.float32),
        mesh=mesh,
        scratch_types=[
            pltpu.VMEM_SHARED((npad, h), jnp.float32),
            pltpu.VMEM((_CH,), jnp.int32),
            pltpu.VMEM((_CH,), jnp.int32),
            pltpu.VMEM((_CH,), jnp.int32),
            pltpu.VMEM((_CH,), jnp.int32),
            pltpu.VMEM((_CH, h), jnp.float32),
            pltpu.VMEM((_CH, h), jnp.float32),
            pltpu.SemaphoreType.DMA,
            pltpu.SemaphoreType.DMA,
            pltpu.SemaphoreType.DMA,
            pltpu.SemaphoreType.DMA,
            pltpu.SemaphoreType.DMA,
            pltpu.SemaphoreType.DMA,
        ],
    )
    def kern(table_h, src_h, dst_h, zeros_h, out_h, acc, srcb0, srcb1,
             dstb0, dstb1, rows0, rows1, ss0, ss1, sd0, sd1, sg0, sg1):
        c = lax.axis_index("c")
        s = lax.axis_index("s")
        wid = c * _NSUB + s
        pltpu.sync_copy(zeros_h.at[pl.ds(s * rps, rps)],
                        acc.at[pl.ds(s * rps, rps)])
        plsc.subcore_barrier()

        srcb = (srcb0, srcb1)
        dstb = (dstb0, dstb1)
        rows = (rows0, rows1)
        ss = (ss0, ss1)
        sd = (sd0, sd1)
        sg = (sg0, sg1)

        def src_load(i, p, wait):
            cp = pltpu.make_async_copy(src_h.at[wid, i], srcb[p], ss[p])
            cp.wait() if wait else cp.start()

        def dst_load(i, p, wait):
            cp = pltpu.make_async_copy(dst_h.at[wid, i], dstb[p], sd[p])
            cp.wait() if wait else cp.start()

        def gather(i, p, wait):
            cp = pltpu.make_async_copy(table_h.at[srcb[p]], rows[p], sg[p])
            cp.wait() if wait else cp.start()

        src_load(0, 0, False)
        src_load(0, 0, True)
        gather(0, 0, False)
        src_load(1, 1, False)
        dst_load(0, 0, False)
        dst_load(1, 1, False)

        def step(i, p, loads):
            src_load(i + 1, 1 - p, True)
            gather(i + 1, 1 - p, False)
            gather(i, p, True)
            if loads:
                src_load(i + 2, p, False)
            dst_load(i, p, True)
            pltpu.sync_copy(rows[p], acc.at[dstb[p]], add=True)
            if loads:
                dst_load(i + 2, p, False)

        def pair(g, carry):
            step(2 * g, 0, True)
            step(2 * g + 1, 1, True)
            return carry

        lax.fori_loop(0, (nchunk - 2) // 2, pair, 0)
        step(nchunk - 2, 0, False)
        gather(nchunk - 1, 1, True)
        dst_load(nchunk - 1, 1, True)
        pltpu.sync_copy(rows[1], acc.at[dstb[1]], add=True)

        plsc.subcore_barrier()
        pltpu.sync_copy(acc.at[pl.ds(s * rps, rps)],
                        out_h.at[c, pl.ds(s * rps, rps)])

    src3 = src.reshape(_NW, nchunk, _CH)
    dst3 = dst.reshape(_NW, nchunk, _CH)
    return kern(table, src3, dst3, zeros)


_BN = 400


def _tc_a_body(x_ref, w1_ref, b1_ref, w0_ref, b0_ref, h_ref, out0_ref):
    hv = jnp.maximum(jnp.dot(x_ref[...], w1_ref[...], **_MM) + b1_ref[...],
                     0.0)
    h_ref[...] = hv
    out0_ref[...] = jnp.dot(hv, w0_ref[...], **_MM) + b0_ref[...]


def _tc_b_body(h_ref, degp_ref, hp_ref, dinvb_ref):
    deg = degp_ref[0] + degp_ref[1]
    dtot = jnp.max(deg, axis=-1, keepdims=True) + 1.0
    dinv = lax.rsqrt(dtot)
    hp_ref[...] = h_ref[...] * dinv
    dinvb_ref[...] = jnp.broadcast_to(dinv, dinvb_ref.shape)


def _tc_c_body(sp_ref, hp_ref, dinvb_ref, w_ref, b_ref, out_ref, hnextp_ref):
    ssum = sp_ref[0] + sp_ref[1] + hp_ref[...]
    dinv = dinvb_ref[...]
    h1 = dinv * ssum
    out_ref[...] = jnp.dot(h1, w_ref[...], **_MM) + b_ref[...]
    hnextp_ref[...] = dinv * h1


def _tc_d_body(sp_ref, h1p_ref, dinvb_ref, w2_ref, b2_ref, out0_ref,
               out1_ref, v0_ref, v1_ref, v2_ref, bl2_ref, logp_ref):
    h2 = dinvb_ref[...] * (sp_ref[0] + sp_ref[1] + h1p_ref[...])
    out2 = jnp.dot(h2, w2_ref[...], **_MM) + b2_ref[...]
    z = (jnp.dot(jnp.maximum(out0_ref[...], 0.0), v0_ref[...], **_MM)
         + jnp.dot(jnp.maximum(out1_ref[...], 0.0), v1_ref[...], **_MM)
         + jnp.dot(jnp.maximum(out2, 0.0), v2_ref[...], **_MM)
         + bl2_ref[...])
    m = jnp.max(z, axis=-1, keepdims=True)
    zs = z - m
    logp_ref[...] = zs - jnp.log(jnp.sum(jnp.exp(zs), axis=-1, keepdims=True))


def _row_spec(width):
    return pl.BlockSpec((_BN, width), lambda i: (i, 0))


def _full_spec(shape):
    nd = len(shape)
    return pl.BlockSpec(shape, lambda i: (0,) * nd)


def _part_spec(width):
    return pl.BlockSpec((_NCORE, _BN, width), lambda i: (0, i, 0))


def kernel(x, edge_index, W_lin1, b_lin1, W0, b0, W1, b1, W2, b2, W_lin2,
           b_lin2):
    n, f = x.shape
    hdim = W_lin1.shape[1]
    cdim = W_lin2.shape[1]
    grid = (n // _BN,)

    e = edge_index.shape[1]
    npadrows = _NPAD - n
    pad_src = jnp.zeros((_EPAD - e,), jnp.int32)
    pad_dst = n + (jnp.arange(_EPAD - e, dtype=jnp.int32) % npadrows)
    src = jnp.concatenate([edge_index[0], pad_src])
    dst = jnp.concatenate([edge_index[1], pad_dst])
    zerosh = jnp.zeros((_NPAD, hdim), jnp.float32)
    onesh = jnp.ones((_CH, hdim), jnp.float32)
    b1r = b_lin1.reshape(1, hdim)
    b0r = b0.reshape(1, hdim)
    b1wr = b1.reshape(1, hdim)
    b2r = b2.reshape(1, hdim)
    bl2r = b_lin2.reshape(1, cdim)
    v0, v1, v2 = (W_lin2[0:hdim], W_lin2[hdim:2 * hdim],
                  W_lin2[2 * hdim:3 * hdim])

    degp = _sc_degree(dst, zerosh, onesh)

    h, out0 = pl.pallas_call(
        _tc_a_body,
        grid=grid,
        in_specs=[_row_spec(f), _full_spec((f, hdim)), _full_spec((1, hdim)),
                  _full_spec((hdim, hdim)), _full_spec((1, hdim))],
        out_specs=[_row_spec(hdim), _row_spec(hdim)],
        out_shape=[jax.ShapeDtypeStruct((n, hdim), jnp.float32)] * 2,
    )(x, W_lin1, b1r, W0, b0r)

    hp, dinvb = pl.pallas_call(
        _tc_b_body,
        grid=grid,
        in_specs=[_row_spec(hdim), _part_spec(hdim)],
        out_specs=[_row_spec(hdim), _row_spec(hdim)],
        out_shape=[jax.ShapeDtypeStruct((n, hdim), jnp.float32)] * 2,
    )(h, degp)

    s1p = _sc_propagate(hp, src, dst, zerosh)

    out1, h1p = pl.pallas_call(
        _tc_c_body,
        grid=grid,
        in_specs=[_part_spec(hdim), _row_spec(hdim), _row_spec(hdim),
                  _full_spec((hdim, hdim)), _full_spec((1, hdim))],
        out_specs=[_row_spec(hdim), _row_spec(hdim)],
        out_shape=[jax.ShapeDtypeStruct((n, hdim), jnp.float32)] * 2,
    )(s1p, hp, dinvb, W1, b1wr)

    s2p = _sc_propagate(h1p, src, dst, zerosh)

    logp = pl.pallas_call(
        _tc_d_body,
        grid=grid,
        in_specs=[_part_spec(hdim), _row_spec(hdim), _row_spec(hdim),
                  _full_spec((hdim, hdim)), _full_spec((1, hdim)),
                  _row_spec(hdim), _row_spec(hdim),
                  _full_spec((hdim, cdim)), _full_spec((hdim, cdim)),
                  _full_spec((hdim, cdim)), _full_spec((1, cdim))],
        out_specs=_row_spec(cdim),
        out_shape=jax.ShapeDtypeStruct((n, cdim), jnp.float32),
    )(s2p, h1p, dinvb, W2, b2r, out0, out1, v0, v1, v2, bl2r)

    return logp


# speedup vs baseline: 2.8769x; 2.8769x over previous
import functools

import jax
import jax.numpy as jnp
from jax import lax
from jax.experimental import pallas as pl
from jax.experimental.pallas import tpu as pltpu
from jax.experimental.pallas import tpu_sc as plsc

_NCORE = 2
_NSUB = 16
_NW = _NCORE * _NSUB
_LANE = 16

_CH = 128
_NPAD = 10240
_EPAD = 327680

_MM = dict(preferred_element_type=jnp.float32, precision=lax.Precision.HIGHEST)


def _sc_degree(dst, zeros, ones):
    nchunk = dst.shape[1]
    npad, h = zeros.shape
    rps = npad // _NSUB

    mesh = plsc.VectorSubcoreMesh(
        core_axis_name="c", subcore_axis_name="s",
        num_cores=_NCORE, num_subcores=_NSUB)

    @functools.partial(
        pl.kernel,
        out_type=jax.ShapeDtypeStruct((_NCORE, npad, h), jnp.float32),
        mesh=mesh,
        scratch_types=[
            pltpu.VMEM_SHARED((npad, h), jnp.float32),
            pltpu.VMEM((_CH,), jnp.int32),
            pltpu.VMEM((_CH,), jnp.int32),
            pltpu.VMEM((_CH, h), jnp.float32),
            pltpu.SemaphoreType.DMA,
            pltpu.SemaphoreType.DMA,
        ],
    )
    def kern(dst_h, zeros_h, ones_h, out_h, acc, dstb0, dstb1, onesv,
             sd0, sd1):
        c = lax.axis_index("c")
        s = lax.axis_index("s")
        wid = c * _NSUB + s
        pltpu.sync_copy(zeros_h.at[pl.ds(s * rps, rps)],
                        acc.at[pl.ds(s * rps, rps)])
        pltpu.sync_copy(ones_h, onesv)
        plsc.subcore_barrier()

        dstb = (dstb0, dstb1)
        sd = (sd0, sd1)

        def dst_load(i, p, wait):
            cp = pltpu.make_async_copy(dst_h.at[wid, i], dstb[p], sd[p])
            cp.wait() if wait else cp.start()

        dst_load(0, 0, False)
        dst_load(1, 1, False)

        def step(i, p, loads):
            dst_load(i, p, True)
            pltpu.sync_copy(onesv, acc.at[dstb[p]], add=True)
            if loads:
                dst_load(i + 2, p, False)

        def pair(g, carry):
            step(2 * g, 0, True)
            step(2 * g + 1, 1, True)
            return carry

        lax.fori_loop(0, (nchunk - 2) // 2, pair, 0)
        step(nchunk - 2, 0, False)
        step(nchunk - 1, 1, False)

        plsc.subcore_barrier()
        pltpu.sync_copy(acc.at[pl.ds(s * rps, rps)],
                        out_h.at[c, pl.ds(s * rps, rps)])

    return kern(dst, zeros, ones)


def _sc_propagate(table, src, dst, zeros):
    _, h = table.shape
    npad = zeros.shape[0]
    nchunk = src.shape[1]
    rps = npad // _NSUB

    mesh = plsc.VectorSubcoreMesh(
        core_axis_name="c", subcore_axis_name="s",
        num_cores=_NCORE, num_subcores=_NSUB)

    @functools.partial(
        pl.kernel,
        out_type=jax.ShapeDtypeStruct((_NCORE, npad, h), jnp.float32),
        mesh=mesh,
        scratch_types=[
            pltpu.VMEM_SHARED((npad, h), jnp.float32),
            pltpu.VMEM((_CH,), jnp.int32),
            pltpu.VMEM((_CH,), jnp.int32),
            pltpu.VMEM((_CH,), jnp.int32),
            pltpu.VMEM((_CH,), jnp.int32),
            pltpu.VMEM((_CH, h), jnp.float32),
            pltpu.VMEM((_CH, h), jnp.float32),
            pltpu.SemaphoreType.DMA,
            pltpu.SemaphoreType.DMA,
            pltpu.SemaphoreType.DMA,
            pltpu.SemaphoreType.DMA,
            pltpu.SemaphoreType.DMA,
            pltpu.SemaphoreType.DMA,
        ],
    )
    def kern(table_h, src_h, dst_h, zeros_h, out_h, acc, srcb0, srcb1,
             dstb0, dstb1, rows0, rows1, ss0, ss1, sd0, sd1, sg0, sg1):
        c = lax.axis_index("c")
        s = lax.axis_index("s")
        wid = c * _NSUB + s
        pltpu.sync_copy(zeros_h.at[pl.ds(s * rps, rps)],
                        acc.at[pl.ds(s * rps, rps)])
        plsc.subcore_barrier()

        srcb = (srcb0, srcb1)
        dstb = (dstb0, dstb1)
        rows = (rows0, rows1)
        ss = (ss0, ss1)
        sd = (sd0, sd1)
        sg = (sg0, sg1)

        def src_load(i, p, wait):
            cp = pltpu.make_async_copy(src_h.at[wid, i], srcb[p], ss[p])
            cp.wait() if wait else cp.start()

        def dst_load(i, p, wait):
            cp = pltpu.make_async_copy(dst_h.at[wid, i], dstb[p], sd[p])
            cp.wait() if wait else cp.start()

        def gather(i, p, wait):
            cp = pltpu.make_async_copy(table_h.at[srcb[p]], rows[p], sg[p])
            cp.wait() if wait else cp.start()

        src_load(0, 0, False)
        src_load(0, 0, True)
        gather(0, 0, False)
        src_load(1, 1, False)
        dst_load(0, 0, False)
        dst_load(1, 1, False)

        def step(i, p, loads):
            src_load(i + 1, 1 - p, True)
            gather(i + 1, 1 - p, False)
            gather(i, p, True)
            if loads:
                src_load(i + 2, p, False)
            dst_load(i, p, True)
            pltpu.sync_copy(rows[p], acc.at[dstb[p]], add=True)
            if loads:
                dst_load(i + 2, p, False)

        def pair(g, carry):
            step(2 * g, 0, True)
            step(2 * g + 1, 1, True)
            return carry

        lax.fori_loop(0, (nchunk - 2) // 2, pair, 0)
        step(nchunk - 2, 0, False)
        gather(nchunk - 1, 1, True)
        dst_load(nchunk - 1, 1, True)
        pltpu.sync_copy(rows[1], acc.at[dstb[1]], add=True)

        plsc.subcore_barrier()
        pltpu.sync_copy(acc.at[pl.ds(s * rps, rps)],
                        out_h.at[c, pl.ds(s * rps, rps)])

    return kern(table, src, dst, zeros)


_BN = 400


def _tc_a_body(x_ref, w1_ref, b1_ref, w0_ref, b0_ref, h_ref, out0_ref):
    hv = jnp.maximum(jnp.dot(x_ref[...], w1_ref[...], **_MM) + b1_ref[...],
                     0.0)
    h_ref[...] = hv
    out0_ref[...] = jnp.dot(hv, w0_ref[...], **_MM) + b0_ref[...]


def _tc_b_body(h_ref, degp_ref, hp_ref, dinvb_ref):
    deg = degp_ref[0] + degp_ref[1]
    dtot = jnp.max(deg, axis=-1, keepdims=True) + 1.0
    dinv = lax.rsqrt(dtot)
    hp_ref[...] = h_ref[...] * dinv
    dinvb_ref[...] = jnp.broadcast_to(dinv, dinvb_ref.shape)


def _tc_c_body(sp_ref, hp_ref, dinvb_ref, w_ref, b_ref, out_ref, hnextp_ref):
    ssum = sp_ref[0] + sp_ref[1] + hp_ref[...]
    dinv = dinvb_ref[...]
    h1 = dinv * ssum
    out_ref[...] = jnp.dot(h1, w_ref[...], **_MM) + b_ref[...]
    hnextp_ref[...] = dinv * h1


def _tc_d_body(sp_ref, h1p_ref, dinvb_ref, w2_ref, b2_ref, out0_ref,
               out1_ref, v0_ref, v1_ref, v2_ref, bl2_ref, logp_ref):
    h2 = dinvb_ref[...] * (sp_ref[0] + sp_ref[1] + h1p_ref[...])
    out2 = jnp.dot(h2, w2_ref[...], **_MM) + b2_ref[...]
    z = (jnp.dot(jnp.maximum(out0_ref[...], 0.0), v0_ref[...], **_MM)
         + jnp.dot(jnp.maximum(out1_ref[...], 0.0), v1_ref[...], **_MM)
         + jnp.dot(jnp.maximum(out2, 0.0), v2_ref[...], **_MM)
         + bl2_ref[...])
    m = jnp.max(z, axis=-1, keepdims=True)
    zs = z - m
    logp_ref[...] = zs - jnp.log(jnp.sum(jnp.exp(zs), axis=-1, keepdims=True))


def _row_spec(width):
    return pl.BlockSpec((_BN, width), lambda i: (i, 0))


def _full_spec(shape):
    nd = len(shape)
    return pl.BlockSpec(shape, lambda i: (0,) * nd)


def _part_spec(width):
    return pl.BlockSpec((_NCORE, _BN, width), lambda i: (0, i, 0))


def kernel(x, edge_index, W_lin1, b_lin1, W0, b0, W1, b1, W2, b2, W_lin2,
           b_lin2):
    n, f = x.shape
    hdim = W_lin1.shape[1]
    cdim = W_lin2.shape[1]
    grid = (n // _BN,)

    e = edge_index.shape[1]
    epw_real = e // _NW
    ppw = (_EPAD - e) // _NW
    npadrows = _NPAD - n
    pad_src = jnp.broadcast_to(
        jnp.arange(ppw, dtype=jnp.int32) % n, (_NW, ppw))
    pad_dst = jnp.broadcast_to(
        n + (jnp.arange(ppw, dtype=jnp.int32) % npadrows), (_NW, ppw))
    nchunk = (_EPAD // _NW) // _CH
    src3 = jnp.concatenate(
        [edge_index[0].reshape(_NW, epw_real), pad_src],
        axis=1).reshape(_NW, nchunk, _CH)
    dst3 = jnp.concatenate(
        [edge_index[1].reshape(_NW, epw_real), pad_dst],
        axis=1).reshape(_NW, nchunk, _CH)
    zerosh = jnp.zeros((_NPAD, hdim), jnp.float32)
    onesh = jnp.ones((_CH, hdim), jnp.float32)
    b1r = b_lin1.reshape(1, hdim)
    b0r = b0.reshape(1, hdim)
    b1wr = b1.reshape(1, hdim)
    b2r = b2.reshape(1, hdim)
    bl2r = b_lin2.reshape(1, cdim)
    v0, v1, v2 = (W_lin2[0:hdim], W_lin2[hdim:2 * hdim],
                  W_lin2[2 * hdim:3 * hdim])

    degp = _sc_degree(dst3, zerosh, onesh)

    h, out0 = pl.pallas_call(
        _tc_a_body,
        grid=grid,
        in_specs=[_row_spec(f), _full_spec((f, hdim)), _full_spec((1, hdim)),
                  _full_spec((hdim, hdim)), _full_spec((1, hdim))],
        out_specs=[_row_spec(hdim), _row_spec(hdim)],
        out_shape=[jax.ShapeDtypeStruct((n, hdim), jnp.float32)] * 2,
    )(x, W_lin1, b1r, W0, b0r)

    hp, dinvb = pl.pallas_call(
        _tc_b_body,
        grid=grid,
        in_specs=[_row_spec(hdim), _part_spec(hdim)],
        out_specs=[_row_spec(hdim), _row_spec(hdim)],
        out_shape=[jax.ShapeDtypeStruct((n, hdim), jnp.float32)] * 2,
    )(h, degp)

    s1p = _sc_propagate(hp, src3, dst3, zerosh)

    out1, h1p = pl.pallas_call(
        _tc_c_body,
        grid=grid,
        in_specs=[_part_spec(hdim), _row_spec(hdim), _row_spec(hdim),
                  _full_spec((hdim, hdim)), _full_spec((1, hdim))],
        out_specs=[_row_spec(hdim), _row_spec(hdim)],
        out_shape=[jax.ShapeDtypeStruct((n, hdim), jnp.float32)] * 2,
    )(s1p, hp, dinvb, W1, b1wr)

    s2p = _sc_propagate(h1p, src3, dst3, zerosh)

    logp = pl.pallas_call(
        _tc_d_body,
        grid=grid,
        in_specs=[_part_spec(hdim), _row_spec(hdim), _row_spec(hdim),
                  _full_spec((hdim, hdim)), _full_spec((1, hdim)),
                  _row_spec(hdim), _row_spec(hdim),
                  _full_spec((hdim, cdim)), _full_spec((hdim, cdim)),
                  _full_spec((hdim, cdim)), _full_spec((1, cdim))],
        out_specs=_row_spec(cdim),
        out_shape=jax.ShapeDtypeStruct((n, cdim), jnp.float32),
    )(s2p, h1p, dinvb, W2, b2r, out0, out1, v0, v1, v2, bl2r)

    return logp


# speedup vs baseline: 2.9093x; 1.0113x over previous
import functools

import jax
import jax.numpy as jnp
from jax import lax
from jax.experimental import pallas as pl
from jax.experimental.pallas import tpu as pltpu
from jax.experimental.pallas import tpu_sc as plsc

_NCORE = 2
_NSUB = 16
_NW = _NCORE * _NSUB
_LANE = 16

_CH = 128
_NPAD = 10240
_EPAD = 327680

_MM = dict(preferred_element_type=jnp.float32)


def _sc_degree(dst, zeros, ones):
    nchunk = dst.shape[1]
    npad, h = zeros.shape
    rps = npad // _NSUB

    mesh = plsc.VectorSubcoreMesh(
        core_axis_name="c", subcore_axis_name="s",
        num_cores=_NCORE, num_subcores=_NSUB)

    @functools.partial(
        pl.kernel,
        out_type=jax.ShapeDtypeStruct((_NCORE, npad, h), jnp.float32),
        mesh=mesh,
        scratch_types=[
            pltpu.VMEM_SHARED((npad, h), jnp.float32),
            pltpu.VMEM((_CH,), jnp.int32),
            pltpu.VMEM((_CH,), jnp.int32),
            pltpu.VMEM((_CH, h), jnp.float32),
            pltpu.SemaphoreType.DMA,
            pltpu.SemaphoreType.DMA,
        ],
    )
    def kern(dst_h, zeros_h, ones_h, out_h, acc, dstb0, dstb1, onesv,
             sd0, sd1):
        c = lax.axis_index("c")
        s = lax.axis_index("s")
        wid = c * _NSUB + s
        pltpu.sync_copy(zeros_h.at[pl.ds(s * rps, rps)],
                        acc.at[pl.ds(s * rps, rps)])
        pltpu.sync_copy(ones_h, onesv)
        plsc.subcore_barrier()

        dstb = (dstb0, dstb1)
        sd = (sd0, sd1)

        def dst_load(i, p, wait):
            cp = pltpu.make_async_copy(dst_h.at[wid, i], dstb[p], sd[p])
            cp.wait() if wait else cp.start()

        dst_load(0, 0, False)
        dst_load(1, 1, False)

        def step(i, p, loads):
            dst_load(i, p, True)
            pltpu.sync_copy(onesv, acc.at[dstb[p]], add=True)
            if loads:
                dst_load(i + 2, p, False)

        def pair(g, carry):
            step(2 * g, 0, True)
            step(2 * g + 1, 1, True)
            return carry

        lax.fori_loop(0, (nchunk - 2) // 2, pair, 0)
        step(nchunk - 2, 0, False)
        step(nchunk - 1, 1, False)

        plsc.subcore_barrier()
        pltpu.sync_copy(acc.at[pl.ds(s * rps, rps)],
                        out_h.at[c, pl.ds(s * rps, rps)])

    return kern(dst, zeros, ones)


def _sc_propagate(table, src, dst, zeros):
    _, h = table.shape
    npad = zeros.shape[0]
    nchunk = src.shape[1]
    rps = npad // _NSUB

    mesh = plsc.VectorSubcoreMesh(
        core_axis_name="c", subcore_axis_name="s",
        num_cores=_NCORE, num_subcores=_NSUB)

    @functools.partial(
        pl.kernel,
        out_type=jax.ShapeDtypeStruct((_NCORE, npad, h), jnp.float32),
        mesh=mesh,
        scratch_types=[
            pltpu.VMEM_SHARED((npad, h), jnp.float32),
            pltpu.VMEM((_CH,), jnp.int32),
            pltpu.VMEM((_CH,), jnp.int32),
            pltpu.VMEM((_CH,), jnp.int32),
            pltpu.VMEM((_CH,), jnp.int32),
            pltpu.VMEM((_CH, h), jnp.float32),
            pltpu.VMEM((_CH, h), jnp.float32),
            pltpu.SemaphoreType.DMA,
            pltpu.SemaphoreType.DMA,
            pltpu.SemaphoreType.DMA,
            pltpu.SemaphoreType.DMA,
            pltpu.SemaphoreType.DMA,
            pltpu.SemaphoreType.DMA,
        ],
    )
    def kern(table_h, src_h, dst_h, zeros_h, out_h, acc, srcb0, srcb1,
             dstb0, dstb1, rows0, rows1, ss0, ss1, sd0, sd1, sg0, sg1):
        c = lax.axis_index("c")
        s = lax.axis_index("s")
        wid = c * _NSUB + s
        pltpu.sync_copy(zeros_h.at[pl.ds(s * rps, rps)],
                        acc.at[pl.ds(s * rps, rps)])
        plsc.subcore_barrier()

        srcb = (srcb0, srcb1)
        dstb = (dstb0, dstb1)
        rows = (rows0, rows1)
        ss = (ss0, ss1)
        sd = (sd0, sd1)
        sg = (sg0, sg1)

        def src_load(i, p, wait):
            cp = pltpu.make_async_copy(src_h.at[wid, i], srcb[p], ss[p])
            cp.wait() if wait else cp.start()

        def dst_load(i, p, wait):
            cp = pltpu.make_async_copy(dst_h.at[wid, i], dstb[p], sd[p])
            cp.wait() if wait else cp.start()

        def gather(i, p, wait):
            cp = pltpu.make_async_copy(table_h.at[srcb[p]], rows[p], sg[p])
            cp.wait() if wait else cp.start()

        src_load(0, 0, False)
        src_load(0, 0, True)
        gather(0, 0, False)
        src_load(1, 1, False)
        dst_load(0, 0, False)
        dst_load(1, 1, False)

        def step(i, p, loads):
            src_load(i + 1, 1 - p, True)
            gather(i + 1, 1 - p, False)
            gather(i, p, True)
            if loads:
                src_load(i + 2, p, False)
            dst_load(i, p, True)
            pltpu.sync_copy(rows[p], acc.at[dstb[p]], add=True)
            if loads:
                dst_load(i + 2, p, False)

        def pair(g, carry):
            step(2 * g, 0, True)
            step(2 * g + 1, 1, True)
            return carry

        lax.fori_loop(0, (nchunk - 2) // 2, pair, 0)
        step(nchunk - 2, 0, False)
        gather(nchunk - 1, 1, True)
        dst_load(nchunk - 1, 1, True)
        pltpu.sync_copy(rows[1], acc.at[dstb[1]], add=True)

        plsc.subcore_barrier()
        pltpu.sync_copy(acc.at[pl.ds(s * rps, rps)],
                        out_h.at[c, pl.ds(s * rps, rps)])

    return kern(table, src, dst, zeros)


_BN = 400


def _tc_ab_body(x_ref, w1_ref, b1_ref, w0_ref, b0_ref, degp_ref, out0_ref,
                hp_ref, dinvb_ref):
    hv = jnp.maximum(jnp.dot(x_ref[...], w1_ref[...], **_MM) + b1_ref[...],
                     0.0)
    out0_ref[...] = jnp.dot(hv, w0_ref[...], **_MM) + b0_ref[...]
    deg = degp_ref[0] + degp_ref[1]
    dtot = jnp.max(deg, axis=-1, keepdims=True) + 1.0
    dinv = lax.rsqrt(dtot)
    hp_ref[...] = hv * dinv
    dinvb_ref[...] = jnp.broadcast_to(dinv, dinvb_ref.shape)


def _tc_c_body(sp_ref, hp_ref, dinvb_ref, w_ref, b_ref, out_ref, hnextp_ref):
    ssum = sp_ref[0] + sp_ref[1] + hp_ref[...]
    dinv = dinvb_ref[...]
    h1 = dinv * ssum
    out_ref[...] = jnp.dot(h1, w_ref[...], **_MM) + b_ref[...]
    hnextp_ref[...] = dinv * h1


def _tc_d_body(sp_ref, h1p_ref, dinvb_ref, w2_ref, b2_ref, out0_ref,
               out1_ref, v0_ref, v1_ref, v2_ref, bl2_ref, logp_ref):
    h2 = dinvb_ref[...] * (sp_ref[0] + sp_ref[1] + h1p_ref[...])
    out2 = jnp.dot(h2, w2_ref[...], **_MM) + b2_ref[...]
    z = (jnp.dot(jnp.maximum(out0_ref[...], 0.0), v0_ref[...], **_MM)
         + jnp.dot(jnp.maximum(out1_ref[...], 0.0), v1_ref[...], **_MM)
         + jnp.dot(jnp.maximum(out2, 0.0), v2_ref[...], **_MM)
         + bl2_ref[...])
    m = jnp.max(z, axis=-1, keepdims=True)
    zs = z - m
    logp_ref[...] = zs - jnp.log(jnp.sum(jnp.exp(zs), axis=-1, keepdims=True))


def _row_spec(width):
    return pl.BlockSpec((_BN, width), lambda i: (i, 0))


def _full_spec(shape):
    nd = len(shape)
    return pl.BlockSpec(shape, lambda i: (0,) * nd)


def _part_spec(width):
    return pl.BlockSpec((_NCORE, _BN, width), lambda i: (0, i, 0))


def kernel(x, edge_index, W_lin1, b_lin1, W0, b0, W1, b1, W2, b2, W_lin2,
           b_lin2):
    n, f = x.shape
    hdim = W_lin1.shape[1]
    cdim = W_lin2.shape[1]
    grid = (n // _BN,)

    e = edge_index.shape[1]
    epw_real = e // _NW
    ppw = (_EPAD - e) // _NW
    npadrows = _NPAD - n
    pad_src = jnp.broadcast_to(
        jnp.arange(ppw, dtype=jnp.int32) % n, (_NW, ppw))
    pad_dst = jnp.broadcast_to(
        n + (jnp.arange(ppw, dtype=jnp.int32) % npadrows), (_NW, ppw))
    nchunk = (_EPAD // _NW) // _CH
    src3 = jnp.concatenate(
        [edge_index[0].reshape(_NW, epw_real), pad_src],
        axis=1).reshape(_NW, nchunk, _CH)
    dst3 = jnp.concatenate(
        [edge_index[1].reshape(_NW, epw_real), pad_dst],
        axis=1).reshape(_NW, nchunk, _CH)
    zerosh = jnp.zeros((_NPAD, hdim), jnp.float32)
    onesh = jnp.ones((_CH, hdim), jnp.float32)
    b1r = b_lin1.reshape(1, hdim)
    b0r = b0.reshape(1, hdim)
    b1wr = b1.reshape(1, hdim)
    b2r = b2.reshape(1, hdim)
    bl2r = b_lin2.reshape(1, cdim)
    v0, v1, v2 = (W_lin2[0:hdim], W_lin2[hdim:2 * hdim],
                  W_lin2[2 * hdim:3 * hdim])

    degp = _sc_degree(dst3, zerosh, onesh)

    out0, hp, dinvb = pl.pallas_call(
        _tc_ab_body,
        grid=grid,
        in_specs=[_row_spec(f), _full_spec((f, hdim)), _full_spec((1, hdim)),
                  _full_spec((hdim, hdim)), _full_spec((1, hdim)),
                  _part_spec(hdim)],
        out_specs=[_row_spec(hdim), _row_spec(hdim), _row_spec(hdim)],
        out_shape=[jax.ShapeDtypeStruct((n, hdim), jnp.float32)] * 3,
    )(x, W_lin1, b1r, W0, b0r, degp)

    s1p = _sc_propagate(hp, src3, dst3, zerosh)

    out1, h1p = pl.pallas_call(
        _tc_c_body,
        grid=grid,
        in_specs=[_part_spec(hdim), _row_spec(hdim), _row_spec(hdim),
                  _full_spec((hdim, hdim)), _full_spec((1, hdim))],
        out_specs=[_row_spec(hdim), _row_spec(hdim)],
        out_shape=[jax.ShapeDtypeStruct((n, hdim), jnp.float32)] * 2,
    )(s1p, hp, dinvb, W1, b1wr)

    s2p = _sc_propagate(h1p, src3, dst3, zerosh)

    logp = pl.pallas_call(
        _tc_d_body,
        grid=grid,
        in_specs=[_part_spec(hdim), _row_spec(hdim), _row_spec(hdim),
                  _full_spec((hdim, hdim)), _full_spec((1, hdim)),
                  _row_spec(hdim), _row_spec(hdim),
                  _full_spec((hdim, cdim)), _full_spec((hdim, cdim)),
                  _full_spec((hdim, cdim)), _full_spec((1, cdim))],
        out_specs=_row_spec(cdim),
        out_shape=jax.ShapeDtypeStruct((n, cdim), jnp.float32),
    )(s2p, h1p, dinvb, W2, b2r, out0, out1, v0, v1, v2, bl2r)

    return logp


# speedup vs baseline: 3.1847x; 1.0947x over previous
import functools

import jax
import jax.numpy as jnp
from jax import lax
from jax.experimental import pallas as pl
from jax.experimental.pallas import tpu as pltpu
from jax.experimental.pallas import tpu_sc as plsc

_NCORE = 2
_NSUB = 16
_NW = _NCORE * _NSUB
_LANE = 16

_CH = 128
_NPAD = 10240
_EPAD = 327680

_MM = dict(preferred_element_type=jnp.float32)


def _sc_degree(dst, zeros, ones):
    nchunk = dst.shape[1]
    npad, h = zeros.shape
    rps = npad // _NSUB

    mesh = plsc.VectorSubcoreMesh(
        core_axis_name="c", subcore_axis_name="s",
        num_cores=_NCORE, num_subcores=_NSUB)

    @functools.partial(
        pl.kernel,
        out_type=jax.ShapeDtypeStruct((_NCORE, npad, h), jnp.float32),
        mesh=mesh,
        scratch_types=[
            pltpu.VMEM_SHARED((npad, h), jnp.float32),
            pltpu.VMEM((_CH,), jnp.int32),
            pltpu.VMEM((_CH,), jnp.int32),
            pltpu.VMEM((_CH, h), jnp.float32),
            pltpu.SemaphoreType.DMA,
            pltpu.SemaphoreType.DMA,
            pltpu.SemaphoreType.DMA,
            pltpu.SemaphoreType.DMA,
        ],
    )
    def kern(dst_h, zeros_h, ones_h, out_h, acc, dstb0, dstb1, onesv,
             sd0, sd1, ssc0, ssc1):
        c = lax.axis_index("c")
        s = lax.axis_index("s")
        wid = c * _NSUB + s
        pltpu.sync_copy(zeros_h.at[pl.ds(s * rps, rps)],
                        acc.at[pl.ds(s * rps, rps)])
        pltpu.sync_copy(ones_h, onesv)
        plsc.subcore_barrier()

        dstb = (dstb0, dstb1)
        sd = (sd0, sd1)

        sc = (ssc0, ssc1)

        def dst_load(i, p, wait):
            cp = pltpu.make_async_copy(dst_h.at[wid, i], dstb[p], sd[p])
            cp.wait() if wait else cp.start()

        def scatter(p, wait):
            if wait:
                pltpu.make_async_copy(onesv, acc.at[dstb[p]], sc[p]).wait()
            else:
                pltpu.async_copy(onesv, acc.at[dstb[p]], sc[p], add=True)

        dst_load(0, 0, False)

        def step(i, p, scwait):
            if scwait:
                scatter(1 - p, True)
            dst_load(i + 1, 1 - p, False)
            dst_load(i, p, True)
            scatter(p, False)

        def pair(g, carry):
            step(2 * g + 1, 1, True)
            step(2 * g + 2, 0, True)
            return carry

        step(0, 0, False)
        lax.fori_loop(0, (nchunk - 2) // 2, pair, 0)
        dst_load(nchunk - 1, 1, True)
        scatter(1, False)
        scatter(0, True)
        scatter(1, True)

        plsc.subcore_barrier()
        pltpu.sync_copy(acc.at[pl.ds(s * rps, rps)],
                        out_h.at[c, pl.ds(s * rps, rps)])

    return kern(dst, zeros, ones)


def _sc_propagate(table, src, dst, zeros):
    _, h = table.shape
    npad = zeros.shape[0]
    nchunk = src.shape[1]
    rps = npad // _NSUB

    mesh = plsc.VectorSubcoreMesh(
        core_axis_name="c", subcore_axis_name="s",
        num_cores=_NCORE, num_subcores=_NSUB)

    @functools.partial(
        pl.kernel,
        out_type=jax.ShapeDtypeStruct((_NCORE, npad, h), jnp.float32),
        mesh=mesh,
        scratch_types=[
            pltpu.VMEM_SHARED((npad, h), jnp.float32),
            pltpu.VMEM((_CH,), jnp.int32),
            pltpu.VMEM((_CH,), jnp.int32),
            pltpu.VMEM((_CH,), jnp.int32),
            pltpu.VMEM((_CH,), jnp.int32),
            pltpu.VMEM((_CH, h), jnp.float32),
            pltpu.VMEM((_CH, h), jnp.float32),
            pltpu.SemaphoreType.DMA,
            pltpu.SemaphoreType.DMA,
            pltpu.SemaphoreType.DMA,
            pltpu.SemaphoreType.DMA,
            pltpu.SemaphoreType.DMA,
            pltpu.SemaphoreType.DMA,
            pltpu.SemaphoreType.DMA,
            pltpu.SemaphoreType.DMA,
        ],
    )
    def kern(table_h, src_h, dst_h, zeros_h, out_h, acc, srcb0, srcb1,
             dstb0, dstb1, rows0, rows1, ss0, ss1, sd0, sd1, sg0, sg1,
             ssc0, ssc1):
        c = lax.axis_index("c")
        s = lax.axis_index("s")
        wid = c * _NSUB + s
        pltpu.sync_copy(zeros_h.at[pl.ds(s * rps, rps)],
                        acc.at[pl.ds(s * rps, rps)])
        plsc.subcore_barrier()

        srcb = (srcb0, srcb1)
        dstb = (dstb0, dstb1)
        rows = (rows0, rows1)
        ss = (ss0, ss1)
        sd = (sd0, sd1)
        sg = (sg0, sg1)
        sc = (ssc0, ssc1)

        def src_load(i, p, wait):
            cp = pltpu.make_async_copy(src_h.at[wid, i], srcb[p], ss[p])
            cp.wait() if wait else cp.start()

        def dst_load(i, p, wait):
            cp = pltpu.make_async_copy(dst_h.at[wid, i], dstb[p], sd[p])
            cp.wait() if wait else cp.start()

        def gather(i, p, wait):
            cp = pltpu.make_async_copy(table_h.at[srcb[p]], rows[p], sg[p])
            cp.wait() if wait else cp.start()

        def scatter(p, wait):
            if wait:
                pltpu.make_async_copy(rows[p], acc.at[dstb[p]], sc[p]).wait()
            else:
                pltpu.async_copy(rows[p], acc.at[dstb[p]], sc[p], add=True)

        src_load(0, 0, False)
        src_load(0, 0, True)
        gather(0, 0, False)
        src_load(1, 1, False)
        dst_load(0, 0, False)

        def step(i, p, srcload2, scwait):
            src_load(i + 1, 1 - p, True)
            if scwait:
                scatter(1 - p, True)
            gather(i + 1, 1 - p, False)
            dst_load(i + 1, 1 - p, False)
            gather(i, p, True)
            if srcload2:
                src_load(i + 2, p, False)
            dst_load(i, p, True)
            scatter(p, False)

        def pair(g, carry):
            step(2 * g + 1, 1, True, True)
            step(2 * g + 2, 0, True, True)
            return carry

        step(0, 0, True, False)
        lax.fori_loop(0, (nchunk - 4) // 2, pair, 0)
        step(nchunk - 3, 1, True, True)
        step(nchunk - 2, 0, False, True)
        gather(nchunk - 1, 1, True)
        dst_load(nchunk - 1, 1, True)
        scatter(1, False)
        scatter(0, True)
        scatter(1, True)

        plsc.subcore_barrier()
        pltpu.sync_copy(acc.at[pl.ds(s * rps, rps)],
                        out_h.at[c, pl.ds(s * rps, rps)])

    return kern(table, src, dst, zeros)


_BN = 2000


def _tc_ab_body(x_ref, w1_ref, b1_ref, w0_ref, b0_ref, degp_ref, out0_ref,
                hp_ref, dinvb_ref):
    hv = jnp.maximum(jnp.dot(x_ref[...], w1_ref[...], **_MM) + b1_ref[...],
                     0.0)
    out0_ref[...] = jnp.dot(hv, w0_ref[...], **_MM) + b0_ref[...]
    deg = degp_ref[0] + degp_ref[1]
    dtot = jnp.max(deg, axis=-1, keepdims=True) + 1.0
    dinv = lax.rsqrt(dtot)
    hp_ref[...] = hv * dinv
    dinvb_ref[...] = jnp.broadcast_to(dinv, dinvb_ref.shape)


def _tc_c_body(sp_ref, hp_ref, dinvb_ref, w_ref, b_ref, out_ref, hnextp_ref):
    ssum = sp_ref[0] + sp_ref[1] + hp_ref[...]
    dinv = dinvb_ref[...]
    h1 = dinv * ssum
    out_ref[...] = jnp.dot(h1, w_ref[...], **_MM) + b_ref[...]
    hnextp_ref[...] = dinv * h1


def _tc_d_body(sp_ref, h1p_ref, dinvb_ref, w2_ref, b2_ref, out0_ref,
               out1_ref, v0_ref, v1_ref, v2_ref, bl2_ref, logp_ref):
    h2 = dinvb_ref[...] * (sp_ref[0] + sp_ref[1] + h1p_ref[...])
    out2 = jnp.dot(h2, w2_ref[...], **_MM) + b2_ref[...]
    z = (jnp.dot(jnp.maximum(out0_ref[...], 0.0), v0_ref[...], **_MM)
         + jnp.dot(jnp.maximum(out1_ref[...], 0.0), v1_ref[...], **_MM)
         + jnp.dot(jnp.maximum(out2, 0.0), v2_ref[...], **_MM)
         + bl2_ref[...])
    m = jnp.max(z, axis=-1, keepdims=True)
    zs = z - m
    logp_ref[...] = zs - jnp.log(jnp.sum(jnp.exp(zs), axis=-1, keepdims=True))


def _row_spec(width):
    return pl.BlockSpec((_BN, width), lambda i: (i, 0))


def _full_spec(shape):
    nd = len(shape)
    return pl.BlockSpec(shape, lambda i: (0,) * nd)


def _part_spec(width):
    return pl.BlockSpec((_NCORE, _BN, width), lambda i: (0, i, 0))


def kernel(x, edge_index, W_lin1, b_lin1, W0, b0, W1, b1, W2, b2, W_lin2,
           b_lin2):
    n, f = x.shape
    hdim = W_lin1.shape[1]
    cdim = W_lin2.shape[1]
    grid = (n // _BN,)

    e = edge_index.shape[1]
    epw_real = e // _NW
    ppw = (_EPAD - e) // _NW
    npadrows = _NPAD - n
    pad_src = jnp.broadcast_to(
        jnp.arange(ppw, dtype=jnp.int32) % n, (_NW, ppw))
    pad_dst = jnp.broadcast_to(
        n + (jnp.arange(ppw, dtype=jnp.int32) % npadrows), (_NW, ppw))
    nchunk = (_EPAD // _NW) // _CH
    src3 = jnp.concatenate(
        [edge_index[0].reshape(_NW, epw_real), pad_src],
        axis=1).reshape(_NW, nchunk, _CH)
    dst3 = jnp.concatenate(
        [edge_index[1].reshape(_NW, epw_real), pad_dst],
        axis=1).reshape(_NW, nchunk, _CH)
    zerosh = jnp.zeros((_NPAD, hdim), jnp.float32)
    onesh = jnp.ones((_CH, hdim), jnp.float32)
    b1r = b_lin1.reshape(1, hdim)
    b0r = b0.reshape(1, hdim)
    b1wr = b1.reshape(1, hdim)
    b2r = b2.reshape(1, hdim)
    bl2r = b_lin2.reshape(1, cdim)
    v0, v1, v2 = (W_lin2[0:hdim], W_lin2[hdim:2 * hdim],
                  W_lin2[2 * hdim:3 * hdim])

    degp = _sc_degree(dst3, zerosh, onesh)

    out0, hp, dinvb = pl.pallas_call(
        _tc_ab_body,
        grid=grid,
        in_specs=[_row_spec(f), _full_spec((f, hdim)), _full_spec((1, hdim)),
                  _full_spec((hdim, hdim)), _full_spec((1, hdim)),
                  _part_spec(hdim)],
        out_specs=[_row_spec(hdim), _row_spec(hdim), _row_spec(hdim)],
        out_shape=[jax.ShapeDtypeStruct((n, hdim), jnp.float32)] * 3,
    )(x, W_lin1, b1r, W0, b0r, degp)

    s1p = _sc_propagate(hp, src3, dst3, zerosh)

    out1, h1p = pl.pallas_call(
        _tc_c_body,
        grid=grid,
        in_specs=[_part_spec(hdim), _row_spec(hdim), _row_spec(hdim),
                  _full_spec((hdim, hdim)), _full_spec((1, hdim))],
        out_specs=[_row_spec(hdim), _row_spec(hdim)],
        out_shape=[jax.ShapeDtypeStruct((n, hdim), jnp.float32)] * 2,
    )(s1p, hp, dinvb, W1, b1wr)

    s2p = _sc_propagate(h1p, src3, dst3, zerosh)

    logp = pl.pallas_call(
        _tc_d_body,
        grid=grid,
        in_specs=[_part_spec(hdim), _row_spec(hdim), _row_spec(hdim),
                  _full_spec((hdim, hdim)), _full_spec((1, hdim)),
                  _row_spec(hdim), _row_spec(hdim),
                  _full_spec((hdim, cdim)), _full_spec((hdim, cdim)),
                  _full_spec((hdim, cdim)), _full_spec((1, cdim))],
        out_specs=_row_spec(cdim),
        out_shape=jax.ShapeDtypeStruct((n, cdim), jnp.float32),
    )(s2p, h1p, dinvb, W2, b2r, out0, out1, v0, v1, v2, bl2r)

    return logp
